# baseline (speedup 1.0000x reference)
import functools

import jax
import jax.numpy as jnp
from jax import lax
from jax.experimental import pallas as pl
from jax.experimental.pallas import tpu as pltpu
from jax.experimental.pallas import tpu_sc as plsc

B = 16384
D = 64
NROWS = 1000000
L = 16
NC = 2
NS = 16
NW = NC * NS
BPW = B // NW
CHUNK = 256

QBLK = 8192
TGRID = -(-NROWS // QBLK)


def _transpose_body(x_ref, y_ref):
    x = x_ref[...]
    eye = jnp.eye(D, dtype=jnp.float32)
    y_ref[...] = lax.dot_general(
        x, eye, (((0,), (0,)), ((), ())),
        precision=lax.Precision.HIGHEST)


_tc_transpose = pl.pallas_call(
    _transpose_body,
    grid=(TGRID,),
    in_specs=[pl.BlockSpec((D, QBLK), lambda i: (0, i))],
    out_specs=pl.BlockSpec((QBLK, D), lambda i: (i, 0)),
    out_shape=jax.ShapeDtypeStruct((NROWS, D), jnp.float32),
)


def _cosine_body(uidx_hbm, iidx_hbm, utab_hbm, itab_hbm, out_hbm,
                 uidx_v, iidx_v, urows_v, irows_v, out_v, usem, isem):
    wid = lax.axis_index("s") * NC + lax.axis_index("c")
    base = wid * BPW

    pltpu.sync_copy(uidx_hbm.at[pl.ds(base, BPW)], uidx_v)
    pltpu.sync_copy(iidx_hbm.at[pl.ds(base, BPW)], iidx_v)

    def chunk_body(ck, _):
        def fire(blk, _):
            uvec = uidx_v[pl.ds(ck * CHUNK + blk * L, L)]
            ivec = iidx_v[pl.ds(ck * CHUNK + blk * L, L)]
            for j in range(L):
                pltpu.async_copy(
                    utab_hbm.at[pl.ds(uvec[j], 1)],
                    urows_v.at[pl.ds(blk * L + j, 1)], usem)
                pltpu.async_copy(
                    itab_hbm.at[pl.ds(ivec[j], 1)],
                    irows_v.at[pl.ds(blk * L + j, 1)], isem)
            return 0

        lax.fori_loop(0, CHUNK // L, fire, 0)
        pltpu.make_async_copy(
            utab_hbm.at[pl.ds(0, CHUNK)], urows_v, usem).wait()
        pltpu.make_async_copy(
            itab_hbm.at[pl.ds(0, CHUNK)], irows_v, isem).wait()

        def block_body(blk, _):
            row_ids = blk * L + lax.iota(jnp.int32, L)

            def d_body(dd, carry):
                dot, uu, ii = carry
                col = jnp.full((L,), dd, jnp.int32)
                u = plsc.load_gather(urows_v, [row_ids, col])
                v = plsc.load_gather(irows_v, [row_ids, col])
                return (dot + u * v, uu + u * u, ii + v * v)

            z = jnp.zeros((L,), jnp.float32)
            dot, uu, ii = lax.fori_loop(0, D, d_body, (z, z, z))

            p = jnp.maximum(uu * ii, 1e-30)
            bits = plsc.bitcast(p, jnp.int32)
            seed = jnp.full((L,), 0x5F3759DF, jnp.int32) - lax.shift_right_logical(
                bits, jnp.full((L,), 1, jnp.int32))
            y = plsc.bitcast(seed, jnp.float32)
            for _ in range(3):
                y = y * (1.5 - 0.5 * p * y * y)
            s = p * y
            denom = jnp.maximum(s, 1e-8)
            out_v[pl.ds(ck * CHUNK + blk * L, L)] = dot / denom
            return 0

        lax.fori_loop(0, CHUNK // L, block_body, 0)
        return 0

    lax.fori_loop(0, BPW // CHUNK, chunk_body, 0)
    pltpu.sync_copy(out_v, out_hbm.at[pl.ds(base, BPW)])


@jax.jit
def _run(query_users, query_items, user_table, item_table):
    u_rows_tab = _tc_transpose(user_table.T)
    i_rows_tab = _tc_transpose(item_table.T)

    mesh = plsc.VectorSubcoreMesh(core_axis_name="c", subcore_axis_name="s")
    k = functools.partial(
        pl.kernel,
        mesh=mesh,
        compiler_params=pltpu.CompilerParams(needs_layout_passes=False),
        out_type=jax.ShapeDtypeStruct((B,), jnp.float32),
        scratch_types=[
            pltpu.VMEM((BPW,), jnp.int32),
            pltpu.VMEM((BPW,), jnp.int32),
            pltpu.VMEM((CHUNK, D), jnp.float32),
            pltpu.VMEM((CHUNK, D), jnp.float32),
            pltpu.VMEM((BPW,), jnp.float32),
            pltpu.SemaphoreType.DMA,
            pltpu.SemaphoreType.DMA,
        ],
    )(_cosine_body)
    return k(query_users, query_items, u_rows_tab, i_rows_tab)


def kernel(query_users, query_items, user_table, item_table):
    qu = query_users.astype(jnp.int32)
    qi = query_items.astype(jnp.int32)
    return _run(qu, qi, user_table, item_table)


# speedup vs baseline: 1.5560x; 1.5560x over previous
import functools

import jax
import jax.numpy as jnp
from jax import lax
from jax.experimental import pallas as pl
from jax.experimental.pallas import tpu as pltpu
from jax.experimental.pallas import tpu_sc as plsc

B = 16384
D = 64
NROWS = 1000000
L = 16
NC = 2
NS = 16
NW = NC * NS
BPW = B // NW
CHUNK = 256

QBLK = 8192
TGRID = -(-NROWS // QBLK)


def _transpose_body(x_ref, y_ref):
    x = x_ref[...]
    eye = jnp.eye(D, dtype=jnp.float32)
    y_ref[...] = lax.dot_general(
        x, eye, (((0,), (0,)), ((), ())),
        precision=lax.Precision.DEFAULT)


_tc_transpose = pl.pallas_call(
    _transpose_body,
    grid=(TGRID,),
    in_specs=[pl.BlockSpec((D, QBLK), lambda i: (0, i))],
    out_specs=pl.BlockSpec((QBLK, D), lambda i: (i, 0)),
    out_shape=jax.ShapeDtypeStruct((NROWS, D), jnp.float32),
)


def _cosine_body(uidx_hbm, iidx_hbm, utab_hbm, itab_hbm, out_hbm,
                 uidx_v, iidx_v, urows_v, irows_v, out_v, usem, isem):
    wid = lax.axis_index("s") * NC + lax.axis_index("c")
    base = wid * BPW

    pltpu.sync_copy(uidx_hbm.at[pl.ds(base, BPW)], uidx_v)
    pltpu.sync_copy(iidx_hbm.at[pl.ds(base, BPW)], iidx_v)

    def chunk_body(ck, _):
        def fire(blk, _):
            uvec = uidx_v[pl.ds(ck * CHUNK + blk * L, L)]
            ivec = iidx_v[pl.ds(ck * CHUNK + blk * L, L)]
            for j in range(L):
                pltpu.async_copy(
                    utab_hbm.at[pl.ds(uvec[j], 1)],
                    urows_v.at[pl.ds(blk * L + j, 1)], usem)
                pltpu.async_copy(
                    itab_hbm.at[pl.ds(ivec[j], 1)],
                    irows_v.at[pl.ds(blk * L + j, 1)], isem)
            return 0

        lax.fori_loop(0, CHUNK // L, fire, 0)
        pltpu.make_async_copy(
            utab_hbm.at[pl.ds(0, CHUNK)], urows_v, usem).wait()
        pltpu.make_async_copy(
            itab_hbm.at[pl.ds(0, CHUNK)], irows_v, isem).wait()

        def block_body(blk, _):
            row_ids = blk * L + lax.iota(jnp.int32, L)

            def d_body(dd, carry):
                dot, uu, ii = carry
                col = jnp.full((L,), dd, jnp.int32)
                u = plsc.load_gather(urows_v, [row_ids, col])
                v = plsc.load_gather(irows_v, [row_ids, col])
                return (dot + u * v, uu + u * u, ii + v * v)

            z = jnp.zeros((L,), jnp.float32)
            dot, uu, ii = lax.fori_loop(0, D, d_body, (z, z, z))

            p = jnp.maximum(uu * ii, 1e-30)
            bits = plsc.bitcast(p, jnp.int32)
            seed = jnp.full((L,), 0x5F3759DF, jnp.int32) - lax.shift_right_logical(
                bits, jnp.full((L,), 1, jnp.int32))
            y = plsc.bitcast(seed, jnp.float32)
            for _ in range(3):
                y = y * (1.5 - 0.5 * p * y * y)
            s = p * y
            denom = jnp.maximum(s, 1e-8)
            out_v[pl.ds(ck * CHUNK + blk * L, L)] = dot / denom
            return 0

        lax.fori_loop(0, CHUNK // L, block_body, 0)
        return 0

    lax.fori_loop(0, BPW // CHUNK, chunk_body, 0)
    pltpu.sync_copy(out_v, out_hbm.at[pl.ds(base, BPW)])


@jax.jit
def _run(query_users, query_items, user_table, item_table):
    u_rows_tab = _tc_transpose(user_table.T)
    i_rows_tab = _tc_transpose(item_table.T)

    mesh = plsc.VectorSubcoreMesh(core_axis_name="c", subcore_axis_name="s")
    k = functools.partial(
        pl.kernel,
        mesh=mesh,
        compiler_params=pltpu.CompilerParams(needs_layout_passes=False),
        out_type=jax.ShapeDtypeStruct((B,), jnp.float32),
        scratch_types=[
            pltpu.VMEM((BPW,), jnp.int32),
            pltpu.VMEM((BPW,), jnp.int32),
            pltpu.VMEM((CHUNK, D), jnp.float32),
            pltpu.VMEM((CHUNK, D), jnp.float32),
            pltpu.VMEM((BPW,), jnp.float32),
            pltpu.SemaphoreType.DMA,
            pltpu.SemaphoreType.DMA,
        ],
    )(_cosine_body)
    return k(query_users, query_items, u_rows_tab, i_rows_tab)


def kernel(query_users, query_items, user_table, item_table):
    qu = query_users.astype(jnp.int32)
    qi = query_items.astype(jnp.int32)
    return _run(qu, qi, user_table, item_table)
